# Initial kernel scaffold; baseline (speedup 1.0000x reference)
import functools

import jax
import jax.numpy as jnp
from jax import lax
from jax.experimental import pallas as pl
from jax.experimental.pallas import tpu as pltpu
from jax.experimental.pallas import tpu_sc as plsc

U_NUM = 2000
I_NUM = 8000
N_NODES = 10000
D = 128
TMP = 0.2
BIAS = 0.0001
E_ADJ = 320000
NC = 2
NS = 16
NW = NC * NS
CHUNK = 128


def _pad_to(x, n):
    return jnp.concatenate([x, jnp.zeros((n - x.shape[0],) + x.shape[1:], x.dtype)])


def _round_up(n, m):
    return ((n + m - 1) // m) * m


_BLK = 400


def _prep_body(cur_ref, ew1_ref, eb1_ref, nw1_ref, nb1_ref, nw2_ref, nb2_ref,
               gn_ref, a0_ref, b0_ref, a1_ref, b1_ref, nm_ref, ns_ref):
    x = cur_ref[...]
    pid = pl.program_id(0)

    @pl.when(pid == 0)
    def _():
        ns_ref[...] = jnp.zeros_like(ns_ref)

    sums = []
    for l in range(2):
        w = ew1_ref[l]
        a = jnp.dot(x, w[:D, :], preferred_element_type=jnp.float32) + eb1_ref[l][None, :]
        b = jnp.dot(x, w[D:, :], preferred_element_type=jnp.float32)
        if l == 0:
            a0_ref[...] = a
            b0_ref[...] = b
        else:
            a1_ref[...] = a
            b1_ref[...] = b
        h = jnp.maximum(
            jnp.dot(x, nw1_ref[l], preferred_element_type=jnp.float32) + nb1_ref[l][None, :], 0.0)
        nmv = jnp.dot(h, nw2_ref[l], preferred_element_type=jnp.float32) + nb2_ref[l][None, :]
        m = jax.nn.sigmoid(gn_ref[:, l:l + 1] + nmv * (1.0 / TMP))
        nm_ref[:, l:l + 1] = m
        sums.append(jnp.sum(m))
    lane = lax.broadcasted_iota(jnp.int32, (1, 128), 1)
    ns_ref[...] += jnp.where(lane == 0, sums[0], 0.0) + jnp.where(lane == 1, sums[1], 0.0)


def _prep(cur, ew1, eb1, nw1, nb1, nw2, nb2, gn):
    grid = N_NODES // _BLK

    def full(*s):
        return pl.BlockSpec(s, lambda i: tuple(0 for _ in s))

    return pl.pallas_call(
        _prep_body,
        grid=(grid,),
        in_specs=[
            pl.BlockSpec((_BLK, D), lambda i: (i, 0)),
            full(2, 2 * D, D), full(2, D), full(2, D, D), full(2, D),
            full(2, D, 1), full(2, 1),
            pl.BlockSpec((_BLK, 2), lambda i: (i, 0)),
        ],
        out_specs=[
            pl.BlockSpec((_BLK, D), lambda i: (i, 0)),
            pl.BlockSpec((_BLK, D), lambda i: (i, 0)),
            pl.BlockSpec((_BLK, D), lambda i: (i, 0)),
            pl.BlockSpec((_BLK, D), lambda i: (i, 0)),
            pl.BlockSpec((_BLK, 2), lambda i: (i, 0)),
            pl.BlockSpec((1, 128), lambda i: (0, 0)),
        ],
        out_shape=[
            jax.ShapeDtypeStruct((N_NODES, D), jnp.float32),
            jax.ShapeDtypeStruct((N_NODES, D), jnp.float32),
            jax.ShapeDtypeStruct((N_NODES, D), jnp.float32),
            jax.ShapeDtypeStruct((N_NODES, D), jnp.float32),
            jax.ShapeDtypeStruct((N_NODES, 2), jnp.float32),
            jax.ShapeDtypeStruct((1, 128), jnp.float32),
        ],
    )(cur, ew1, eb1, nw1, nb1, nw2, nb2, gn)


_MESH = plsc.VectorSubcoreMesh(core_axis_name="c", subcore_axis_name="s")


def _emask_body(a0, b0, a1, b1, rowp, colp, adjp, ge, w2s, b2s,
                nv_out, epart_out,
                idx_r, idx_c, arows, brows, adjv, gev, outv, w2v, b2v, partv,
                sem_a, sem_b):
    cid = lax.axis_index("c")
    sid = lax.axis_index("s")
    wid = sid * NC + cid
    ep = rowp.shape[0]
    ept = ep // NW
    nch = ept // CHUNK
    base = wid * ept
    pltpu.sync_copy(w2s, w2v)
    pltpu.sync_copy(b2s, b2v)
    lane = lax.broadcasted_iota(jnp.int32, (16,), 0)
    for l in range(2):
        at = a0 if l == 0 else a1
        bt = b0 if l == 0 else b1

        def chunk_body(c, reg, l=l, at=at, bt=bt):
            off = base + c * CHUNK
            pltpu.sync_copy(rowp.at[pl.ds(off, CHUNK)], idx_r)
            pltpu.sync_copy(colp.at[pl.ds(off, CHUNK)], idx_c)
            pltpu.sync_copy(adjp.at[pl.ds(off, CHUNK)], adjv)
            pltpu.sync_copy(ge.at[l, pl.ds(off, CHUNK)], gev)
            ca = pltpu.async_copy(at.at[idx_r], arows, sem_a)
            cb = pltpu.async_copy(bt.at[idx_c], brows, sem_b)
            ca.wait()
            cb.wait()
            w2regs = [w2v[l, pl.ds(16 * k, 16)] for k in range(8)]
            b2reg = b2v[l]

            def grp_body(g, reg2):
                gb = g * 16
                dots = jnp.zeros((16,), jnp.float32)
                for e in range(16):
                    acc = jnp.zeros((16,), jnp.float32)
                    for k in range(8):
                        av = arows[gb + e, pl.ds(16 * k, 16)]
                        bv = brows[gb + e, pl.ds(16 * k, 16)]
                        acc = acc + jnp.maximum(av + bv, 0.0) * w2regs[k]
                    dots = jnp.where(lane == e, jnp.sum(acc), dots)
                g16 = gev[pl.ds(gb, 16)]
                ad16 = adjv[pl.ds(gb, 16)]
                m = 1.0 / (1.0 + jnp.exp(-(g16 + dots + b2reg)))
                nvv = ad16 * m
                outv[pl.ds(gb, 16)] = nvv
                return reg2 + nvv

            reg = lax.fori_loop(0, CHUNK // 16, grp_body, reg)
            pltpu.sync_copy(outv, nv_out.at[l, pl.ds(off, CHUNK)])
            return reg

        reg = lax.fori_loop(0, nch, chunk_body, jnp.zeros((16,), jnp.float32))
        partv[...] = reg
        pltpu.sync_copy(partv, epart_out.at[l * NW + wid])


def _emask(a0, b0, a1, b1, rowp, colp, adjp, ge, w2s, b2s):
    ep = rowp.shape[0]
    return pl.kernel(
        _emask_body,
        out_type=[
            jax.ShapeDtypeStruct((2, ep), jnp.float32),
            jax.ShapeDtypeStruct((2 * NW, 16), jnp.float32),
        ],
        mesh=_MESH,
        scratch_types=[
            pltpu.VMEM((CHUNK,), jnp.int32),
            pltpu.VMEM((CHUNK,), jnp.int32),
            pltpu.VMEM((CHUNK, D), jnp.float32),
            pltpu.VMEM((CHUNK, D), jnp.float32),
            pltpu.VMEM((CHUNK,), jnp.float32),
            pltpu.VMEM((CHUNK,), jnp.float32),
            pltpu.VMEM((CHUNK,), jnp.float32),
            pltpu.VMEM((2, D), jnp.float32),
            pltpu.VMEM((2, 16), jnp.float32),
            pltpu.VMEM((16,), jnp.float32),
            pltpu.SemaphoreType.DMA,
            pltpu.SemaphoreType.DMA,
        ],
    )(a0, b0, a1, b1, rowp, colp, adjp, ge, w2s, b2s)


_SR = N_NODES // NS


def _spmm_body(x, rowp, colp, vals, zrows, part,
               rowv, colv, valsv, xrows, stripe, acc, sem):
    cid = lax.axis_index("c")
    sid = lax.axis_index("s")
    wid = sid * NC + cid
    ep = rowp.shape[0]
    ept = ep // NW
    nch = ept // CHUNK
    base = wid * ept
    pltpu.sync_copy(zrows.at[pl.ds(sid * _SR, _SR)], acc.at[pl.ds(sid * _SR, _SR)])
    plsc.subcore_barrier()

    def chunk_body(c, _):
        off = base + c * CHUNK
        pltpu.sync_copy(rowp.at[pl.ds(off, CHUNK)], rowv)
        pltpu.sync_copy(colp.at[pl.ds(off, CHUNK)], colv)
        pltpu.sync_copy(vals.at[pl.ds(off, CHUNK)], valsv)
        pltpu.async_copy(x.at[colv], xrows, sem).wait()

        def grp_body(g, __):
            gb = g * 16
            for e in range(16):
                vb = plsc.load_gather(valsv, [jnp.full((16,), gb + e, jnp.int32)])
                for k in range(8):
                    xrows[gb + e, pl.ds(16 * k, 16)] = xrows[gb + e, pl.ds(16 * k, 16)] * vb
            return 0

        lax.fori_loop(0, CHUNK // 16, grp_body, 0)
        pltpu.sync_copy(xrows, acc.at[rowv], add=True)
        return 0

    lax.fori_loop(0, nch, chunk_body, 0)
    plsc.subcore_barrier()
    pltpu.sync_copy(acc.at[pl.ds(sid * _SR, _SR)], stripe)
    pltpu.sync_copy(stripe, part.at[cid, pl.ds(sid * _SR, _SR)])


def _spmm(x, rowp, colp, vals, zrows):
    return pl.kernel(
        _spmm_body,
        out_type=[jax.ShapeDtypeStruct((NC, N_NODES, D), jnp.float32)],
        mesh=_MESH,
        scratch_types=[
            pltpu.VMEM((CHUNK,), jnp.int32),
            pltpu.VMEM((CHUNK,), jnp.int32),
            pltpu.VMEM((CHUNK,), jnp.float32),
            pltpu.VMEM((CHUNK, D), jnp.float32),
            pltpu.VMEM((_SR, D), jnp.float32),
            pltpu.VMEM_SHARED((N_NODES, D), jnp.float32),
            pltpu.SemaphoreType.DMA,
        ],
    )(x, rowp, colp, vals, zrows)[0]


def _add2_body(a_ref, b_ref, o_ref):
    o_ref[...] = a_ref[...] + b_ref[...]


def _add2(a, b):
    return pl.pallas_call(
        _add2_body,
        grid=(N_NODES // _BLK,),
        in_specs=[pl.BlockSpec((_BLK, D), lambda i: (i, 0))] * 2,
        out_specs=pl.BlockSpec((_BLK, D), lambda i: (i, 0)),
        out_shape=jax.ShapeDtypeStruct((N_NODES, D), jnp.float32),
    )(a, b)


def _combine_body(l, nm_ref, cur_ref, p0_ref, p1_ref, o_ref):
    m = nm_ref[:, l:l + 1]
    o_ref[...] = m * cur_ref[...] + (1.0 - m) * (p0_ref[...] + p1_ref[...])


def _combine(l, nm, cur, mp):
    return pl.pallas_call(
        functools.partial(_combine_body, l),
        grid=(N_NODES // _BLK,),
        in_specs=[
            pl.BlockSpec((_BLK, 2), lambda i: (i, 0)),
            pl.BlockSpec((_BLK, D), lambda i: (i, 0)),
            pl.BlockSpec((_BLK, D), lambda i: (i, 0)),
            pl.BlockSpec((_BLK, D), lambda i: (i, 0)),
        ],
        out_specs=pl.BlockSpec((_BLK, D), lambda i: (i, 0)),
        out_shape=jax.ShapeDtypeStruct((N_NODES, D), jnp.float32),
    )(nm, cur, mp[0], mp[1])


_BQ = 1024
_BT = _BQ // NW


def _fin_body(uid, pid_, nid, e0, e1, e2a, e2b, n1, n2a, n2b, epart, nsum,
              oue, oip, oin, oun, onp, onn, sreg,
              idxv, r0, r1, r2, r3, ob, epbuf, nsbuf, sbuf,
              s0, s1, s2, s3):
    cid = lax.axis_index("c")
    sid = lax.axis_index("s")
    wid = sid * NC + cid
    base = wid * _BT
    third = 1.0 / 3.0

    def gather_combine(ids_hbm, tabs, out_hbm, offset):
        pltpu.sync_copy(ids_hbm.at[pl.ds(base, _BT)], idxv)
        if offset:
            for j in range(_BT // 16):
                idxv[pl.ds(16 * j, 16)] = idxv[pl.ds(16 * j, 16)] + U_NUM
        c0 = pltpu.async_copy(tabs[0].at[idxv], r0, s0)
        c1 = pltpu.async_copy(tabs[1].at[idxv], r1, s1)
        c2 = pltpu.async_copy(tabs[2].at[idxv], r2, s2)
        c3 = pltpu.async_copy(tabs[3].at[idxv], r3, s3)
        c0.wait()
        c1.wait()
        c2.wait()
        c3.wait()
        for j in range(_BT):
            for k in range(8):
                sl = pl.ds(16 * k, 16)
                ob[j, sl] = (r0[j, sl] + r1[j, sl] + r2[j, sl] + r3[j, sl]) * third
        pltpu.sync_copy(ob, out_hbm.at[pl.ds(base, _BT)])

    etabs = (e0, e1, e2a, e2b)
    ntabs = (e0, n1, n2a, n2b)
    gather_combine(uid, etabs, oue, False)
    gather_combine(uid, ntabs, oun, False)
    gather_combine(pid_, etabs, oip, True)
    gather_combine(pid_, ntabs, onp, True)
    gather_combine(nid, etabs, oin, True)
    gather_combine(nid, ntabs, onn, True)

    @pl.when(wid == 0)
    def _():
        pltpu.sync_copy(epart, epbuf)
        pltpu.sync_copy(nsum, nsbuf)

        def rb(i, a):
            return a + epbuf[i]

        acc = lax.fori_loop(0, 2 * NW, rb, jnp.zeros((16,), jnp.float32))
        er = jnp.sum(acc) * (1.0 / (E_ADJ * 2))
        lane = lax.broadcasted_iota(jnp.int32, (16,), 0)
        nr = jnp.sum(jnp.where(lane < 2, nsbuf[0, pl.ds(0, 16)], 0.0)) * (1.0 / (N_NODES * 2))
        sbuf[...] = jnp.where(lane == 0, er, jnp.where(lane == 1, nr, 0.0))
        pltpu.sync_copy(sbuf, sreg)


def _final(uid, pid_, nid, e0, e1, e2a, e2b, n1, n2a, n2b, epart, nsum):
    obt = jax.ShapeDtypeStruct((_BQ, D), jnp.float32)
    return pl.kernel(
        _fin_body,
        out_type=[obt, obt, obt, obt, obt, obt,
                  jax.ShapeDtypeStruct((16,), jnp.float32)],
        mesh=_MESH,
        scratch_types=[
            pltpu.VMEM((_BT,), jnp.int32),
            pltpu.VMEM((_BT, D), jnp.float32),
            pltpu.VMEM((_BT, D), jnp.float32),
            pltpu.VMEM((_BT, D), jnp.float32),
            pltpu.VMEM((_BT, D), jnp.float32),
            pltpu.VMEM((_BT, D), jnp.float32),
            pltpu.VMEM((2 * NW, 16), jnp.float32),
            pltpu.VMEM((1, 128), jnp.float32),
            pltpu.VMEM((16,), jnp.float32),
            pltpu.SemaphoreType.DMA,
            pltpu.SemaphoreType.DMA,
            pltpu.SemaphoreType.DMA,
            pltpu.SemaphoreType.DMA,
        ],
    )(uid, pid_, nid, e0, e1, e2a, e2b, n1, n2a, n2b, epart, nsum)


def kernel(cur_user_emb, cur_item_emb, original_user_emb, original_item_emb,
           user_id, pos_item, neg_item,
           row, col, adj_vals, rw_row, rw_col, rw_vals,
           node_W1, node_b1, node_W2, node_b2,
           edge_W1, edge_b1, edge_W2, edge_b2):
    cur = jnp.concatenate([cur_user_emb, cur_item_emb], axis=0)
    orig = jnp.concatenate([original_user_emb, original_item_emb], axis=0)

    e_rw = rw_row.shape[0]
    ep_adj = _round_up(E_ADJ, NW * CHUNK)
    ep_rw = _round_up(e_rw, NW * CHUNK)

    key = jax.random.key(42)
    ge_l, gn_l = [], []
    for l in range(2):
        u = jax.random.uniform(jax.random.fold_in(key, 2 * l), (E_ADJ, 1))
        eps = (BIAS - (1 - BIAS)) * u + (1 - BIAS)
        ge_l.append(_pad_to((jnp.log(eps) - jnp.log1p(-eps))[:, 0] * (1.0 / TMP), ep_adj))
        u2 = jax.random.uniform(jax.random.fold_in(key, 2 * l + 1), (N_NODES, 1))
        eps2 = (BIAS - (1 - BIAS)) * u2 + (1 - BIAS)
        gn_l.append((jnp.log(eps2) - jnp.log1p(-eps2)) * (1.0 / TMP))
    ge = jnp.stack(ge_l)
    gn = jnp.concatenate(gn_l, axis=1)

    rowp = _pad_to(row, ep_adj)
    colp = _pad_to(col, ep_adj)
    adjp = _pad_to(adj_vals, ep_adj)
    rwrp = _pad_to(rw_row, ep_rw)
    rwcp = _pad_to(rw_col, ep_rw)
    rwvp = _pad_to(rw_vals, ep_rw)

    w2s = edge_W2[:, :, 0] * (1.0 / TMP)
    b2s = jnp.tile(edge_b2 * (1.0 / TMP), (1, 16))
    zrows = jnp.zeros((N_NODES, D), jnp.float32)

    a0, b0, a1, b1, nmask, nsum = _prep(cur, edge_W1, edge_b1, node_W1,
                                        node_b1, node_W2, node_b2, gn)
    nv, epart = _emask(a0, b0, a1, b1, rowp, colp, adjp, ge, w2s, b2s)

    e1p = _spmm(orig, rowp, colp, nv[0], zrows)
    e1 = _add2(e1p[0], e1p[1])
    e2p = _spmm(e1, rowp, colp, nv[1], zrows)

    mp1 = _spmm(orig, rwrp, rwcp, rwvp, zrows)
    n1in = _combine(0, nmask, orig, mp1)
    t1 = _spmm(n1in, rowp, colp, adjp, zrows)
    n1 = _add2(t1[0], t1[1])
    mp2 = _spmm(n1, rwrp, rwcp, rwvp, zrows)
    n2in = _combine(1, nmask, n1, mp2)
    n2p = _spmm(n2in, rowp, colp, adjp, zrows)

    oue, oip, oin, oun, onp_, onn, sreg = _final(
        user_id, pos_item, neg_item,
        orig, e1, e2p[0], e2p[1], n1, n2p[0], n2p[1], epart, nsum)
    return (oue, oip, oin, oun, onp_, onn, sreg[0], sreg[1])


# speedup vs baseline: 2.0387x; 2.0387x over previous
import functools

import jax
import jax.numpy as jnp
from jax import lax
from jax.experimental import pallas as pl
from jax.experimental.pallas import tpu as pltpu
from jax.experimental.pallas import tpu_sc as plsc

U_NUM = 2000
I_NUM = 8000
N_NODES = 10000
D = 128
TMP = 0.2
BIAS = 0.0001
E_ADJ = 320000
NC = 2
NS = 16
NW = NC * NS
CHUNK = 128


def _pad_to(x, n):
    return jnp.concatenate([x, jnp.zeros((n - x.shape[0],) + x.shape[1:], x.dtype)])


def _round_up(n, m):
    return ((n + m - 1) // m) * m


_BLK = 400


def _prep_body(cur_ref, ew1_ref, eb1_ref, nw1_ref, nb1_ref, nw2_ref, nb2_ref,
               gn_ref, a0_ref, b0_ref, a1_ref, b1_ref, nm_ref, ns_ref):
    x = cur_ref[...]
    pid = pl.program_id(0)

    @pl.when(pid == 0)
    def _():
        ns_ref[...] = jnp.zeros_like(ns_ref)

    sums = []
    for l in range(2):
        w = ew1_ref[l]
        a = jnp.dot(x, w[:D, :], preferred_element_type=jnp.float32) + eb1_ref[l][None, :]
        b = jnp.dot(x, w[D:, :], preferred_element_type=jnp.float32)
        if l == 0:
            a0_ref[...] = a
            b0_ref[...] = b
        else:
            a1_ref[...] = a
            b1_ref[...] = b
        h = jnp.maximum(
            jnp.dot(x, nw1_ref[l], preferred_element_type=jnp.float32) + nb1_ref[l][None, :], 0.0)
        nmv = jnp.dot(h, nw2_ref[l], preferred_element_type=jnp.float32) + nb2_ref[l][None, :]
        m = jax.nn.sigmoid(gn_ref[:, l:l + 1] + nmv * (1.0 / TMP))
        nm_ref[:, l:l + 1] = m
        sums.append(jnp.sum(m))
    lane = lax.broadcasted_iota(jnp.int32, (1, 128), 1)
    ns_ref[...] += jnp.where(lane == 0, sums[0], 0.0) + jnp.where(lane == 1, sums[1], 0.0)


def _prep(cur, ew1, eb1, nw1, nb1, nw2, nb2, gn):
    grid = N_NODES // _BLK

    def full(*s):
        return pl.BlockSpec(s, lambda i: tuple(0 for _ in s))

    return pl.pallas_call(
        _prep_body,
        grid=(grid,),
        in_specs=[
            pl.BlockSpec((_BLK, D), lambda i: (i, 0)),
            full(2, 2 * D, D), full(2, D), full(2, D, D), full(2, D),
            full(2, D, 1), full(2, 1),
            pl.BlockSpec((_BLK, 2), lambda i: (i, 0)),
        ],
        out_specs=[
            pl.BlockSpec((_BLK, D), lambda i: (i, 0)),
            pl.BlockSpec((_BLK, D), lambda i: (i, 0)),
            pl.BlockSpec((_BLK, D), lambda i: (i, 0)),
            pl.BlockSpec((_BLK, D), lambda i: (i, 0)),
            pl.BlockSpec((_BLK, 2), lambda i: (i, 0)),
            pl.BlockSpec((1, 128), lambda i: (0, 0)),
        ],
        out_shape=[
            jax.ShapeDtypeStruct((N_NODES, D), jnp.float32),
            jax.ShapeDtypeStruct((N_NODES, D), jnp.float32),
            jax.ShapeDtypeStruct((N_NODES, D), jnp.float32),
            jax.ShapeDtypeStruct((N_NODES, D), jnp.float32),
            jax.ShapeDtypeStruct((N_NODES, 2), jnp.float32),
            jax.ShapeDtypeStruct((1, 128), jnp.float32),
        ],
    )(cur, ew1, eb1, nw1, nb1, nw2, nb2, gn)


@functools.lru_cache(maxsize=None)
def _mesh():
    return plsc.VectorSubcoreMesh(core_axis_name="c", subcore_axis_name="s",
                                  num_cores=NC, num_subcores=NS)


_GDN = lax.GatherDimensionNumbers(offset_dims=(), collapsed_slice_dims=(0,),
                                  start_index_map=(0,))


def _shuf(v, idx16):
    return lax.gather(v, idx16[:, None], _GDN, (1,),
                      mode=lax.GatherScatterMode.PROMISE_IN_BOUNDS)


def _lanesum(v):
    lane = lax.broadcasted_iota(jnp.int32, (16,), 0)
    for sh in (8, 4, 2, 1):
        v = v + _shuf(v, lane ^ sh)
    return v


def _emask_body(a0, b0, a1, b1, rowp, colp, adjp, ge0, ge1, w2s, b2s,
                nv0_out, nv1_out, epart_out,
                idx_r, idx_c, arows, brows, adjv, gev, outv, w2v, b2v, partv,
                sem_a, sem_b):
    cid = lax.axis_index("c")
    sid = lax.axis_index("s")
    wid = sid * NC + cid
    ep = rowp.shape[0]
    ept = ep // NW
    nch = ept // CHUNK
    base = wid * ept
    pltpu.sync_copy(w2s, w2v)
    pltpu.sync_copy(b2s, b2v)
    lane = lax.broadcasted_iota(jnp.int32, (16,), 0)
    for l in range(2):
        at = a0 if l == 0 else a1
        bt = b0 if l == 0 else b1
        gel = ge0 if l == 0 else ge1
        nvl = nv0_out if l == 0 else nv1_out

        def chunk_body(c, reg, l=l, at=at, bt=bt, gel=gel, nvl=nvl):
            off = base + c * CHUNK
            pltpu.sync_copy(rowp.at[pl.ds(off, CHUNK)], idx_r)
            pltpu.sync_copy(colp.at[pl.ds(off, CHUNK)], idx_c)
            pltpu.sync_copy(adjp.at[pl.ds(off, CHUNK)], adjv)
            pltpu.sync_copy(gel.at[pl.ds(off, CHUNK)], gev)
            ca = pltpu.async_copy(at.at[idx_r], arows, sem_a)
            cb = pltpu.async_copy(bt.at[idx_c], brows, sem_b)
            ca.wait()
            cb.wait()
            w2regs = [w2v[l, pl.ds(16 * k, 16)] for k in range(8)]
            b2reg = b2v[l]

            def grp_body(g, reg2):
                gb = g * 16
                dots = jnp.zeros((16,), jnp.float32)
                for e in range(16):
                    acc = jnp.zeros((16,), jnp.float32)
                    for k in range(8):
                        av = arows[gb + e, pl.ds(16 * k, 16)]
                        bv = brows[gb + e, pl.ds(16 * k, 16)]
                        acc = acc + jnp.maximum(av + bv, 0.0) * w2regs[k]
                    dots = jnp.where(lane == e, _lanesum(acc), dots)
                g16 = gev[pl.ds(gb, 16)]
                ad16 = adjv[pl.ds(gb, 16)]
                m = 1.0 / (1.0 + jnp.exp(-(g16 + dots + b2reg)))
                nvv = ad16 * m
                outv[pl.ds(gb, 16)] = nvv
                return reg2 + nvv

            reg = lax.fori_loop(0, CHUNK // 16, grp_body, reg)
            pltpu.sync_copy(outv, nvl.at[pl.ds(off, CHUNK)])
            return reg

        reg = lax.fori_loop(0, nch, chunk_body, jnp.zeros((16,), jnp.float32))
        partv[0, ...] = reg
        pltpu.sync_copy(partv, epart_out.at[l * NW + wid])


def _emask(a0, b0, a1, b1, rowp, colp, adjp, ge0, ge1, w2s, b2s):
    ep = rowp.shape[0]
    return pl.kernel(
        _emask_body,
        out_type=[
            jax.ShapeDtypeStruct((ep,), jnp.float32),
            jax.ShapeDtypeStruct((ep,), jnp.float32),
            jax.ShapeDtypeStruct((2 * NW, 1, 16), jnp.float32),
        ],
        mesh=_mesh(),
        scratch_types=[
            pltpu.VMEM((CHUNK,), jnp.int32),
            pltpu.VMEM((CHUNK,), jnp.int32),
            pltpu.VMEM((CHUNK, D), jnp.float32),
            pltpu.VMEM((CHUNK, D), jnp.float32),
            pltpu.VMEM((CHUNK,), jnp.float32),
            pltpu.VMEM((CHUNK,), jnp.float32),
            pltpu.VMEM((CHUNK,), jnp.float32),
            pltpu.VMEM((2, D), jnp.float32),
            pltpu.VMEM((2, 16), jnp.float32),
            pltpu.VMEM((1, 16), jnp.float32),
            pltpu.SemaphoreType.DMA,
            pltpu.SemaphoreType.DMA,
        ],
    )(a0, b0, a1, b1, rowp, colp, adjp, ge0, ge1, w2s, b2s)


_SR = 640
_SR_LAST = N_NODES - _SR * (NS - 1)
_NACC = _SR * NS


def _spmm_body(x, rowp, colp, vals, zrows, part,
               rowv, colv, valsv, xrows, acc, sem):
    cid = lax.axis_index("c")
    sid = lax.axis_index("s")
    wid = sid * NC + cid
    ep = rowp.shape[0]
    ept = ep // NW
    nch = ept // CHUNK
    base = wid * ept
    rbase = sid * _SR

    @pl.when(sid < NS - 1)
    def _():
        pltpu.sync_copy(zrows.at[pl.ds(rbase, _SR)], acc.at[pl.ds(rbase, _SR)])

    @pl.when(sid == NS - 1)
    def _():
        pltpu.sync_copy(zrows.at[pl.ds(rbase, _SR_LAST)],
                        acc.at[pl.ds(rbase, _SR_LAST)])

    plsc.subcore_barrier()

    def chunk_body(c, _):
        off = base + c * CHUNK
        pltpu.sync_copy(rowp.at[pl.ds(off, CHUNK)], rowv)
        pltpu.sync_copy(colp.at[pl.ds(off, CHUNK)], colv)
        pltpu.sync_copy(vals.at[pl.ds(off, CHUNK)], valsv)
        pltpu.async_copy(x.at[colv], xrows, sem).wait()

        def grp_body(g, __):
            gb = g * 16
            v16 = valsv[pl.ds(gb, 16)]
            for e in range(16):
                vb = v16[e]
                for k in range(8):
                    xrows[gb + e, pl.ds(16 * k, 16)] = xrows[gb + e, pl.ds(16 * k, 16)] * vb
            return 0

        lax.fori_loop(0, CHUNK // 16, grp_body, 0)
        pltpu.sync_copy(xrows, acc.at[rowv], add=True)
        return 0

    lax.fori_loop(0, nch, chunk_body, 0)
    plsc.subcore_barrier()

    nfull = jnp.where(sid < NS - 1, _SR // CHUNK, _SR_LAST // CHUNK)

    def out_chunk(i, _):
        rb2 = rbase + i * CHUNK
        pltpu.sync_copy(acc.at[pl.ds(rb2, CHUNK)], xrows)
        pltpu.sync_copy(xrows, part.at[cid, pl.ds(rb2, CHUNK)])
        return 0

    lax.fori_loop(0, nfull, out_chunk, 0)

    @pl.when(sid == NS - 1)
    def _():
        tail = _SR_LAST - (_SR_LAST // CHUNK) * CHUNK
        tb = rbase + (_SR_LAST // CHUNK) * CHUNK
        pltpu.sync_copy(acc.at[pl.ds(tb, tail)], xrows.at[pl.ds(0, tail)])
        pltpu.sync_copy(xrows.at[pl.ds(0, tail)], part.at[cid, pl.ds(tb, tail)])


def _spmm(x, rowp, colp, vals, zrows):
    return pl.kernel(
        _spmm_body,
        out_type=[jax.ShapeDtypeStruct((NC, N_NODES, D), jnp.float32)],
        mesh=_mesh(),
        scratch_types=[
            pltpu.VMEM((CHUNK,), jnp.int32),
            pltpu.VMEM((CHUNK,), jnp.int32),
            pltpu.VMEM((CHUNK,), jnp.float32),
            pltpu.VMEM((CHUNK, D), jnp.float32),
            pltpu.VMEM_SHARED((_NACC, D), jnp.float32),
            pltpu.SemaphoreType.DMA,
        ],
    )(x, rowp, colp, vals, zrows)[0]


def _add2_body(a_ref, b_ref, o_ref):
    o_ref[...] = a_ref[...] + b_ref[...]


def _add2(a, b):
    return pl.pallas_call(
        _add2_body,
        grid=(N_NODES // _BLK,),
        in_specs=[pl.BlockSpec((_BLK, D), lambda i: (i, 0))] * 2,
        out_specs=pl.BlockSpec((_BLK, D), lambda i: (i, 0)),
        out_shape=jax.ShapeDtypeStruct((N_NODES, D), jnp.float32),
    )(a, b)


def _combine_body(l, nm_ref, cur_ref, p0_ref, p1_ref, o_ref):
    m = nm_ref[:, l:l + 1]
    o_ref[...] = m * cur_ref[...] + (1.0 - m) * (p0_ref[...] + p1_ref[...])


def _combine(l, nm, cur, mp):
    return pl.pallas_call(
        functools.partial(_combine_body, l),
        grid=(N_NODES // _BLK,),
        in_specs=[
            pl.BlockSpec((_BLK, 2), lambda i: (i, 0)),
            pl.BlockSpec((_BLK, D), lambda i: (i, 0)),
            pl.BlockSpec((_BLK, D), lambda i: (i, 0)),
            pl.BlockSpec((_BLK, D), lambda i: (i, 0)),
        ],
        out_specs=pl.BlockSpec((_BLK, D), lambda i: (i, 0)),
        out_shape=jax.ShapeDtypeStruct((N_NODES, D), jnp.float32),
    )(nm, cur, mp[0], mp[1])


_BQ = 1024
_BT = _BQ // NW


def _fin_body(uid, pid_, nid, e0, e1, e2a, e2b, n1, n2a, n2b, epart, nsum,
              oue, oip, oin, oun, onp, onn, sreg,
              idxv, r0, r1, r2, r3, ob, epbuf, nsbuf, sbuf,
              s0, s1, s2, s3):
    cid = lax.axis_index("c")
    sid = lax.axis_index("s")
    wid = sid * NC + cid
    base = wid * _BT
    third = 1.0 / 3.0

    def gather_combine(ids_hbm, tabs, out_hbm, offset):
        pltpu.sync_copy(ids_hbm.at[pl.ds(base, _BT)], idxv)
        if offset:
            for j in range(_BT // 16):
                idxv[pl.ds(16 * j, 16)] = idxv[pl.ds(16 * j, 16)] + U_NUM
        c0 = pltpu.async_copy(tabs[0].at[idxv], r0, s0)
        c1 = pltpu.async_copy(tabs[1].at[idxv], r1, s1)
        c2 = pltpu.async_copy(tabs[2].at[idxv], r2, s2)
        c3 = pltpu.async_copy(tabs[3].at[idxv], r3, s3)
        c0.wait()
        c1.wait()
        c2.wait()
        c3.wait()
        def row_body(j, _):
            for k in range(8):
                sl = pl.ds(16 * k, 16)
                ob[j, sl] = (r0[j, sl] + r1[j, sl] + r2[j, sl] + r3[j, sl]) * third
            return 0

        lax.fori_loop(0, _BT, row_body, 0)
        pltpu.sync_copy(ob, out_hbm.at[pl.ds(base, _BT)])

    etabs = (e0, e1, e2a, e2b)
    ntabs = (e0, n1, n2a, n2b)
    gather_combine(uid, etabs, oue, False)
    gather_combine(uid, ntabs, oun, False)
    gather_combine(pid_, etabs, oip, True)
    gather_combine(pid_, ntabs, onp, True)
    gather_combine(nid, etabs, oin, True)
    gather_combine(nid, ntabs, onn, True)

    @pl.when(wid == 0)
    def _():
        pltpu.sync_copy(epart, epbuf)
        pltpu.sync_copy(nsum, nsbuf)

        def rb(i, a):
            return a + epbuf[i, 0]

        acc = lax.fori_loop(0, 2 * NW, rb, jnp.zeros((16,), jnp.float32))
        er = _lanesum(acc) * (1.0 / (E_ADJ * 2))
        lane = lax.broadcasted_iota(jnp.int32, (16,), 0)
        nr = _lanesum(jnp.where(lane < 2, nsbuf[0, pl.ds(0, 16)], 0.0)) * (1.0 / (N_NODES * 2))
        sbuf[...] = jnp.where(lane == 0, er, jnp.where(lane == 1, nr, 0.0))
        pltpu.sync_copy(sbuf, sreg)


def _final(uid, pid_, nid, e0, e1, e2a, e2b, n1, n2a, n2b, epart, nsum):
    obt = jax.ShapeDtypeStruct((_BQ, D), jnp.float32)
    return pl.kernel(
        _fin_body,
        out_type=[obt, obt, obt, obt, obt, obt,
                  jax.ShapeDtypeStruct((16,), jnp.float32)],
        mesh=_mesh(),
        scratch_types=[
            pltpu.VMEM((_BT,), jnp.int32),
            pltpu.VMEM((_BT, D), jnp.float32),
            pltpu.VMEM((_BT, D), jnp.float32),
            pltpu.VMEM((_BT, D), jnp.float32),
            pltpu.VMEM((_BT, D), jnp.float32),
            pltpu.VMEM((_BT, D), jnp.float32),
            pltpu.VMEM((2 * NW, 1, 16), jnp.float32),
            pltpu.VMEM((1, 128), jnp.float32),
            pltpu.VMEM((16,), jnp.float32),
            pltpu.SemaphoreType.DMA,
            pltpu.SemaphoreType.DMA,
            pltpu.SemaphoreType.DMA,
            pltpu.SemaphoreType.DMA,
        ],
    )(uid, pid_, nid, e0, e1, e2a, e2b, n1, n2a, n2b, epart, nsum)


def kernel(cur_user_emb, cur_item_emb, original_user_emb, original_item_emb,
           user_id, pos_item, neg_item,
           row, col, adj_vals, rw_row, rw_col, rw_vals,
           node_W1, node_b1, node_W2, node_b2,
           edge_W1, edge_b1, edge_W2, edge_b2):
    cur = jnp.concatenate([cur_user_emb, cur_item_emb], axis=0)
    orig = jnp.concatenate([original_user_emb, original_item_emb], axis=0)

    e_rw = rw_row.shape[0]
    ep_adj = _round_up(E_ADJ, NW * CHUNK)
    ep_rw = _round_up(e_rw, NW * CHUNK)

    key = jax.random.key(42)
    ge_l, gn_l = [], []
    for l in range(2):
        u = jax.random.uniform(jax.random.fold_in(key, 2 * l), (E_ADJ, 1))
        eps = (BIAS - (1 - BIAS)) * u + (1 - BIAS)
        ge_l.append(_pad_to((jnp.log(eps) - jnp.log1p(-eps))[:, 0] * (1.0 / TMP), ep_adj))
        u2 = jax.random.uniform(jax.random.fold_in(key, 2 * l + 1), (N_NODES, 1))
        eps2 = (BIAS - (1 - BIAS)) * u2 + (1 - BIAS)
        gn_l.append((jnp.log(eps2) - jnp.log1p(-eps2)) * (1.0 / TMP))
    gn = jnp.concatenate(gn_l, axis=1)

    rowp = _pad_to(row, ep_adj)
    colp = _pad_to(col, ep_adj)
    adjp = _pad_to(adj_vals, ep_adj)
    rwrp = _pad_to(rw_row, ep_rw)
    rwcp = _pad_to(rw_col, ep_rw)
    rwvp = _pad_to(rw_vals, ep_rw)

    w2s = edge_W2[:, :, 0] * (1.0 / TMP)
    b2s = jnp.tile(edge_b2 * (1.0 / TMP), (1, 16))
    zrows = jnp.zeros((N_NODES, D), jnp.float32)

    a0, b0, a1, b1, nmask, nsum = _prep(cur, edge_W1, edge_b1, node_W1,
                                        node_b1, node_W2, node_b2, gn)
    nv0, nv1, epart = _emask(a0, b0, a1, b1, rowp, colp, adjp,
                             ge_l[0], ge_l[1], w2s, b2s)

    e1p = _spmm(orig, rowp, colp, nv0, zrows)
    e1 = _add2(e1p[0], e1p[1])
    e2p = _spmm(e1, rowp, colp, nv1, zrows)

    mp1 = _spmm(orig, rwrp, rwcp, rwvp, zrows)
    n1in = _combine(0, nmask, orig, mp1)
    t1 = _spmm(n1in, rowp, colp, adjp, zrows)
    n1 = _add2(t1[0], t1[1])
    mp2 = _spmm(n1, rwrp, rwcp, rwvp, zrows)
    n2in = _combine(1, nmask, n1, mp2)
    n2p = _spmm(n2in, rowp, colp, adjp, zrows)

    oue, oip, oin, oun, onp_, onn, sreg = _final(
        user_id, pos_item, neg_item,
        orig, e1, e2p[0], e2p[1], n1, n2p[0], n2p[1], epart, nsum)
    return (oue, oip, oin, oun, onp_, onn, sreg[0], sreg[1])


# speedup vs baseline: 2.4903x; 1.2215x over previous
import functools

import jax
import jax.numpy as jnp
from jax import lax
from jax.experimental import pallas as pl
from jax.experimental.pallas import tpu as pltpu
from jax.experimental.pallas import tpu_sc as plsc

U_NUM = 2000
I_NUM = 8000
N_NODES = 10000
D = 128
TMP = 0.2
BIAS = 0.0001
E_ADJ = 320000
NC = 2
NS = 16
NW = NC * NS
CHUNK = 128


def _pad_to(x, n):
    return jnp.concatenate([x, jnp.zeros((n - x.shape[0],) + x.shape[1:], x.dtype)])


def _round_up(n, m):
    return ((n + m - 1) // m) * m


_BLK = 400


def _prep_body(cur_ref, ew1_ref, eb1_ref, nw1_ref, nb1_ref, nw2_ref, nb2_ref,
               gn_ref, a0_ref, b0_ref, a1_ref, b1_ref, nm_ref, ns_ref):
    x = cur_ref[...]
    pid = pl.program_id(0)

    @pl.when(pid == 0)
    def _():
        ns_ref[...] = jnp.zeros_like(ns_ref)

    sums = []
    for l in range(2):
        w = ew1_ref[l]
        a = jnp.dot(x, w[:D, :], preferred_element_type=jnp.float32) + eb1_ref[l][None, :]
        b = jnp.dot(x, w[D:, :], preferred_element_type=jnp.float32)
        if l == 0:
            a0_ref[...] = a
            b0_ref[...] = b
        else:
            a1_ref[...] = a
            b1_ref[...] = b
        h = jnp.maximum(
            jnp.dot(x, nw1_ref[l], preferred_element_type=jnp.float32) + nb1_ref[l][None, :], 0.0)
        nmv = jnp.dot(h, nw2_ref[l], preferred_element_type=jnp.float32) + nb2_ref[l][None, :]
        m = jax.nn.sigmoid(gn_ref[:, l:l + 1] + nmv * (1.0 / TMP))
        nm_ref[:, l:l + 1] = m
        sums.append(jnp.sum(m))
    lane = lax.broadcasted_iota(jnp.int32, (1, 128), 1)
    ns_ref[...] += jnp.where(lane == 0, sums[0], 0.0) + jnp.where(lane == 1, sums[1], 0.0)


def _prep(cur, ew1, eb1, nw1, nb1, nw2, nb2, gn):
    grid = N_NODES // _BLK

    def full(*s):
        return pl.BlockSpec(s, lambda i: tuple(0 for _ in s))

    return pl.pallas_call(
        _prep_body,
        grid=(grid,),
        in_specs=[
            pl.BlockSpec((_BLK, D), lambda i: (i, 0)),
            full(2, 2 * D, D), full(2, D), full(2, D, D), full(2, D),
            full(2, D, 1), full(2, 1),
            pl.BlockSpec((_BLK, 2), lambda i: (i, 0)),
        ],
        out_specs=[
            pl.BlockSpec((_BLK, D), lambda i: (i, 0)),
            pl.BlockSpec((_BLK, D), lambda i: (i, 0)),
            pl.BlockSpec((_BLK, D), lambda i: (i, 0)),
            pl.BlockSpec((_BLK, D), lambda i: (i, 0)),
            pl.BlockSpec((_BLK, 2), lambda i: (i, 0)),
            pl.BlockSpec((1, 128), lambda i: (0, 0)),
        ],
        out_shape=[
            jax.ShapeDtypeStruct((N_NODES, D), jnp.float32),
            jax.ShapeDtypeStruct((N_NODES, D), jnp.float32),
            jax.ShapeDtypeStruct((N_NODES, D), jnp.float32),
            jax.ShapeDtypeStruct((N_NODES, D), jnp.float32),
            jax.ShapeDtypeStruct((N_NODES, 2), jnp.float32),
            jax.ShapeDtypeStruct((1, 128), jnp.float32),
        ],
    )(cur, ew1, eb1, nw1, nb1, nw2, nb2, gn)


@functools.lru_cache(maxsize=None)
def _mesh():
    return plsc.VectorSubcoreMesh(core_axis_name="c", subcore_axis_name="s",
                                  num_cores=NC, num_subcores=NS)


_GDN = lax.GatherDimensionNumbers(offset_dims=(), collapsed_slice_dims=(0,),
                                  start_index_map=(0,))


def _shuf(v, idx16):
    return lax.gather(v, idx16[:, None], _GDN, (1,),
                      mode=lax.GatherScatterMode.PROMISE_IN_BOUNDS)


def _lanesum(v):
    lane = lax.broadcasted_iota(jnp.int32, (16,), 0)
    for sh in (8, 4, 2, 1):
        v = v + _shuf(v, lane ^ sh)
    return v


def _emask_body(a0, b0, a1, b1, rowp, colp, adjp, ge0, ge1, w2s, b2s,
                nv0_out, nv1_out, epart_out,
                idx_r, idx_c, arows, brows, adjv, gev, outv, w2v, b2v, partv,
                sem_a, sem_b):
    cid = lax.axis_index("c")
    sid = lax.axis_index("s")
    wid = sid * NC + cid
    ep = rowp.shape[0]
    ept = ep // NW
    nch = ept // CHUNK
    base = wid * ept
    pltpu.sync_copy(w2s, w2v)
    pltpu.sync_copy(b2s, b2v)
    lane = lax.broadcasted_iota(jnp.int32, (16,), 0)
    for l in range(2):
        at = a0 if l == 0 else a1
        bt = b0 if l == 0 else b1
        gel = ge0 if l == 0 else ge1
        nvl = nv0_out if l == 0 else nv1_out

        def chunk_body(c, reg, l=l, at=at, bt=bt, gel=gel, nvl=nvl):
            off = base + c * CHUNK
            pltpu.sync_copy(rowp.at[pl.ds(off, CHUNK)], idx_r)
            pltpu.sync_copy(colp.at[pl.ds(off, CHUNK)], idx_c)
            pltpu.sync_copy(adjp.at[pl.ds(off, CHUNK)], adjv)
            pltpu.sync_copy(gel.at[pl.ds(off, CHUNK)], gev)
            ca = pltpu.async_copy(at.at[idx_r], arows, sem_a)
            cb = pltpu.async_copy(bt.at[idx_c], brows, sem_b)
            ca.wait()
            cb.wait()
            w2regs = [w2v[l, pl.ds(16 * k, 16)] for k in range(8)]
            b2reg = b2v[l]

            def grp_body(g, reg2):
                gb = g * 16
                dots = jnp.zeros((16,), jnp.float32)
                for e in range(16):
                    acc = jnp.zeros((16,), jnp.float32)
                    for k in range(8):
                        av = arows[gb + e, pl.ds(16 * k, 16)]
                        bv = brows[gb + e, pl.ds(16 * k, 16)]
                        acc = acc + jnp.maximum(av + bv, 0.0) * w2regs[k]
                    dots = jnp.where(lane == e, _lanesum(acc), dots)
                g16 = gev[pl.ds(gb, 16)]
                ad16 = adjv[pl.ds(gb, 16)]
                m = 1.0 / (1.0 + jnp.exp(-(g16 + dots + b2reg)))
                nvv = ad16 * m
                outv[pl.ds(gb, 16)] = nvv
                return reg2 + nvv

            reg = lax.fori_loop(0, CHUNK // 16, grp_body, reg)
            pltpu.sync_copy(outv, nvl.at[pl.ds(off, CHUNK)])
            return reg

        reg = lax.fori_loop(0, nch, chunk_body, jnp.zeros((16,), jnp.float32))
        partv[0, ...] = reg
        pltpu.sync_copy(partv, epart_out.at[l * NW + wid])


def _emask(a0, b0, a1, b1, rowp, colp, adjp, ge0, ge1, w2s, b2s):
    ep = rowp.shape[0]
    return pl.kernel(
        _emask_body,
        out_type=[
            jax.ShapeDtypeStruct((ep,), jnp.float32),
            jax.ShapeDtypeStruct((ep,), jnp.float32),
            jax.ShapeDtypeStruct((2 * NW, 1, 16), jnp.float32),
        ],
        mesh=_mesh(),
        scratch_types=[
            pltpu.VMEM((CHUNK,), jnp.int32),
            pltpu.VMEM((CHUNK,), jnp.int32),
            pltpu.VMEM((CHUNK, D), jnp.float32),
            pltpu.VMEM((CHUNK, D), jnp.float32),
            pltpu.VMEM((CHUNK,), jnp.float32),
            pltpu.VMEM((CHUNK,), jnp.float32),
            pltpu.VMEM((CHUNK,), jnp.float32),
            pltpu.VMEM((2, D), jnp.float32),
            pltpu.VMEM((2, 16), jnp.float32),
            pltpu.VMEM((1, 16), jnp.float32),
            pltpu.SemaphoreType.DMA,
            pltpu.SemaphoreType.DMA,
        ],
    )(a0, b0, a1, b1, rowp, colp, adjp, ge0, ge1, w2s, b2s)


_SR = 640
_SR_LAST = N_NODES - _SR * (NS - 1)
_NACC = _SR * NS
_UACC = 8192
_H_USR = 2000


def _scale_rows(xb, vv):
    def grp_body(g, _):
        gb = g * 16
        v16 = vv[pl.ds(gb, 16)]
        for e in range(16):
            vb = v16[e]
            for k in range(8):
                xb[gb + e, pl.ds(16 * k, 16)] = xb[gb + e, pl.ds(16 * k, 16)] * vb
        return 0

    lax.fori_loop(0, CHUNK // 16, grp_body, 0)


def _spmm_pipe(combo, vals, x, acc, base_blk, nch, cb, vv, xb, sems, remap):
    def load_chunk(q, blk):
        pltpu.sync_copy(combo.at[blk], cb[q])
        pltpu.sync_copy(vals.at[pl.ds(blk * CHUNK, CHUNK)], vv[q])
        if remap is not None:
            for k in range(8):
                sl = pl.ds(16 * k, 16)
                cb[q][0, sl] = cb[q][0, sl] - remap

    def fire(q):
        pltpu.async_copy(x.at[cb[q].at[1]], xb[q], sems[q])

    def drain(q):
        pltpu.make_async_copy(x.at[cb[q].at[1]], xb[q], sems[q]).wait()

    def consume(q):
        _scale_rows(xb[q], vv[q])
        pltpu.sync_copy(xb[q], acc.at[cb[q].at[0]], add=True)

    load_chunk(0, base_blk)
    fire(0)

    def pair_body(i, _):
        for q in (0, 1):
            c = 2 * i + q
            load_chunk(1 - q, base_blk + c + 1)
            fire(1 - q)
            drain(q)
            consume(q)
        return 0

    lax.fori_loop(0, (nch - 1) // 2, pair_body, 0)
    drain(0)
    consume(0)


def _spmm_adj_body(x, combo, vals, zrows, out,
                   cb0, cb1, vv0, vv1, xb0, xb1, acc, s0, s1):
    cid = lax.axis_index("c")
    sid = lax.axis_index("s")
    ep = vals.shape[0]
    ept = ep // NW
    nch = ept // CHUNK
    base_blk = (cid * (ep // 2) + sid * ept) // CHUNK

    @pl.when(cid == 0)
    def _():
        pltpu.sync_copy(zrows.at[pl.ds(0, 128)], acc.at[pl.ds(sid * 128, 128)])

    @pl.when(cid == 1)
    def _():
        pltpu.sync_copy(zrows.at[pl.ds(0, 512)], acc.at[pl.ds(sid * 512, 512)])

    plsc.subcore_barrier()
    _spmm_pipe(combo, vals, x, acc, base_blk, nch,
               [cb0, cb1], [vv0, vv1], [xb0, xb1], [s0, s1],
               remap=cid * _H_USR)
    plsc.subcore_barrier()

    @pl.when(cid == 0)
    def _():
        @pl.when(sid < NS - 1)
        def _():
            pltpu.sync_copy(acc.at[pl.ds(sid * 128, 128)],
                            out.at[pl.ds(sid * 128, 128)])

        @pl.when(sid == NS - 1)
        def _():
            pltpu.sync_copy(acc.at[pl.ds(1920, 80)], out.at[pl.ds(1920, 80)])

    @pl.when(cid == 1)
    def _():
        @pl.when(sid < NS - 1)
        def _():
            pltpu.sync_copy(acc.at[pl.ds(sid * 512, 512)],
                            out.at[pl.ds(_H_USR + sid * 512, 512)])

        @pl.when(sid == NS - 1)
        def _():
            pltpu.sync_copy(acc.at[pl.ds(7680, 320)], out.at[pl.ds(9680, 320)])


def _spmm_adj(x, combo, vals, zrows):
    return pl.kernel(
        _spmm_adj_body,
        out_type=[jax.ShapeDtypeStruct((N_NODES, D), jnp.float32)],
        mesh=_mesh(),
        scratch_types=[
            pltpu.VMEM((2, CHUNK), jnp.int32),
            pltpu.VMEM((2, CHUNK), jnp.int32),
            pltpu.VMEM((CHUNK,), jnp.float32),
            pltpu.VMEM((CHUNK,), jnp.float32),
            pltpu.VMEM((CHUNK, D), jnp.float32),
            pltpu.VMEM((CHUNK, D), jnp.float32),
            pltpu.VMEM_SHARED((_UACC, D), jnp.float32),
            pltpu.SemaphoreType.DMA,
            pltpu.SemaphoreType.DMA,
        ],
    )(x, combo, vals, zrows)[0]


def _spmm_rw_body(x, combo, vals, zrows, part,
                  cb0, cb1, vv0, vv1, xb0, xb1, acc, s0, s1):
    cid = lax.axis_index("c")
    sid = lax.axis_index("s")
    wid = sid * NC + cid
    ep = vals.shape[0]
    ept = ep // NW
    nch = ept // CHUNK
    base_blk = wid * ept // CHUNK
    rbase = sid * _SR

    @pl.when(sid < NS - 1)
    def _():
        pltpu.sync_copy(zrows, acc.at[pl.ds(rbase, _SR)])

    @pl.when(sid == NS - 1)
    def _():
        pltpu.sync_copy(zrows.at[pl.ds(0, _SR_LAST)], acc.at[pl.ds(rbase, _SR_LAST)])

    plsc.subcore_barrier()
    _spmm_pipe(combo, vals, x, acc, base_blk, nch,
               [cb0, cb1], [vv0, vv1], [xb0, xb1], [s0, s1], remap=None)
    plsc.subcore_barrier()

    @pl.when(sid < NS - 1)
    def _():
        pltpu.sync_copy(acc.at[pl.ds(rbase, _SR)], part.at[cid, pl.ds(rbase, _SR)])

    @pl.when(sid == NS - 1)
    def _():
        pltpu.sync_copy(acc.at[pl.ds(rbase, _SR_LAST)],
                        part.at[cid, pl.ds(rbase, _SR_LAST)])


def _spmm_rw(x, combo, vals, zrows):
    return pl.kernel(
        _spmm_rw_body,
        out_type=[jax.ShapeDtypeStruct((NC, N_NODES, D), jnp.float32)],
        mesh=_mesh(),
        scratch_types=[
            pltpu.VMEM((2, CHUNK), jnp.int32),
            pltpu.VMEM((2, CHUNK), jnp.int32),
            pltpu.VMEM((CHUNK,), jnp.float32),
            pltpu.VMEM((CHUNK,), jnp.float32),
            pltpu.VMEM((CHUNK, D), jnp.float32),
            pltpu.VMEM((CHUNK, D), jnp.float32),
            pltpu.VMEM_SHARED((_NACC, D), jnp.float32),
            pltpu.SemaphoreType.DMA,
            pltpu.SemaphoreType.DMA,
        ],
    )(x, combo, vals, zrows)[0]


def _combine_body(l, nm_ref, cur_ref, p0_ref, p1_ref, o_ref):
    m = nm_ref[:, l:l + 1]
    o_ref[...] = m * cur_ref[...] + (1.0 - m) * (p0_ref[...] + p1_ref[...])


def _combine(l, nm, cur, mp):
    return pl.pallas_call(
        functools.partial(_combine_body, l),
        grid=(N_NODES // _BLK,),
        in_specs=[
            pl.BlockSpec((_BLK, 2), lambda i: (i, 0)),
            pl.BlockSpec((_BLK, D), lambda i: (i, 0)),
            pl.BlockSpec((_BLK, D), lambda i: (i, 0)),
            pl.BlockSpec((_BLK, D), lambda i: (i, 0)),
        ],
        out_specs=pl.BlockSpec((_BLK, D), lambda i: (i, 0)),
        out_shape=jax.ShapeDtypeStruct((N_NODES, D), jnp.float32),
    )(nm, cur, mp[0], mp[1])


_BQ = 1024
_BT = _BQ // NW


def _fin_body(uid, pid_, nid, e0, e1, e2, n1, n2, epart, nsum,
              oue, oip, oin, oun, onp, onn, sreg,
              idxv, r0, r1, r2, ob, epbuf, nsbuf, sbuf,
              s0, s1, s2):
    cid = lax.axis_index("c")
    sid = lax.axis_index("s")
    wid = sid * NC + cid
    base = wid * _BT
    third = 1.0 / 3.0

    def gather_combine(ids_hbm, tabs, out_hbm, offset):
        pltpu.sync_copy(ids_hbm.at[pl.ds(base, _BT)], idxv)
        if offset:
            for j in range(_BT // 16):
                idxv[pl.ds(16 * j, 16)] = idxv[pl.ds(16 * j, 16)] + U_NUM
        c0 = pltpu.async_copy(tabs[0].at[idxv], r0, s0)
        c1 = pltpu.async_copy(tabs[1].at[idxv], r1, s1)
        c2 = pltpu.async_copy(tabs[2].at[idxv], r2, s2)
        c0.wait()
        c1.wait()
        c2.wait()
        def row_body(j, _):
            for k in range(8):
                sl = pl.ds(16 * k, 16)
                ob[j, sl] = (r0[j, sl] + r1[j, sl] + r2[j, sl]) * third
            return 0

        lax.fori_loop(0, _BT, row_body, 0)
        pltpu.sync_copy(ob, out_hbm.at[pl.ds(base, _BT)])

    etabs = (e0, e1, e2)
    ntabs = (e0, n1, n2)
    gather_combine(uid, etabs, oue, False)
    gather_combine(uid, ntabs, oun, False)
    gather_combine(pid_, etabs, oip, True)
    gather_combine(pid_, ntabs, onp, True)
    gather_combine(nid, etabs, oin, True)
    gather_combine(nid, ntabs, onn, True)

    @pl.when(wid == 0)
    def _():
        pltpu.sync_copy(epart, epbuf)
        pltpu.sync_copy(nsum, nsbuf)

        def rb(i, a):
            return a + epbuf[i, 0]

        acc = lax.fori_loop(0, 2 * NW, rb, jnp.zeros((16,), jnp.float32))
        er = _lanesum(acc) * (1.0 / (E_ADJ * 2))
        lane = lax.broadcasted_iota(jnp.int32, (16,), 0)
        nr = _lanesum(jnp.where(lane < 2, nsbuf[0, pl.ds(0, 16)], 0.0)) * (1.0 / (N_NODES * 2))
        sbuf[...] = jnp.where(lane == 0, er, jnp.where(lane == 1, nr, 0.0))
        pltpu.sync_copy(sbuf, sreg)


def _final(uid, pid_, nid, e0, e1, e2, n1, n2, epart, nsum):
    obt = jax.ShapeDtypeStruct((_BQ, D), jnp.float32)
    return pl.kernel(
        _fin_body,
        out_type=[obt, obt, obt, obt, obt, obt,
                  jax.ShapeDtypeStruct((16,), jnp.float32)],
        mesh=_mesh(),
        scratch_types=[
            pltpu.VMEM((_BT,), jnp.int32),
            pltpu.VMEM((_BT, D), jnp.float32),
            pltpu.VMEM((_BT, D), jnp.float32),
            pltpu.VMEM((_BT, D), jnp.float32),
            pltpu.VMEM((_BT, D), jnp.float32),
            pltpu.VMEM((2 * NW, 1, 16), jnp.float32),
            pltpu.VMEM((1, 128), jnp.float32),
            pltpu.VMEM((16,), jnp.float32),
            pltpu.SemaphoreType.DMA,
            pltpu.SemaphoreType.DMA,
            pltpu.SemaphoreType.DMA,
        ],
    )(uid, pid_, nid, e0, e1, e2, n1, n2, epart, nsum)


def kernel(cur_user_emb, cur_item_emb, original_user_emb, original_item_emb,
           user_id, pos_item, neg_item,
           row, col, adj_vals, rw_row, rw_col, rw_vals,
           node_W1, node_b1, node_W2, node_b2,
           edge_W1, edge_b1, edge_W2, edge_b2):
    cur = jnp.concatenate([cur_user_emb, cur_item_emb], axis=0)
    orig = jnp.concatenate([original_user_emb, original_item_emb], axis=0)

    e_rw = rw_row.shape[0]
    ep_adj = _round_up(E_ADJ, NW * CHUNK)
    ep_rw = _round_up(e_rw, NW * CHUNK)
    if (ep_rw // NW // CHUNK) % 2 == 0:
        ep_rw += NW * CHUNK

    hh = E_ADJ // 2
    hp = ep_adj // 2
    padu = hp - hh

    def pad_halves(arr, pv1):
        z0 = jnp.zeros((padu,), arr.dtype)
        z1 = jnp.full((padu,), pv1, arr.dtype)
        return jnp.concatenate([arr[:hh], z0, arr[hh:], z1])

    key = jax.random.key(42)
    ge_l, gn_l = [], []
    for l in range(2):
        u = jax.random.uniform(jax.random.fold_in(key, 2 * l), (E_ADJ, 1))
        eps = (BIAS - (1 - BIAS)) * u + (1 - BIAS)
        ge_l.append(pad_halves((jnp.log(eps) - jnp.log1p(-eps))[:, 0] * (1.0 / TMP), 0))
        u2 = jax.random.uniform(jax.random.fold_in(key, 2 * l + 1), (N_NODES, 1))
        eps2 = (BIAS - (1 - BIAS)) * u2 + (1 - BIAS)
        gn_l.append((jnp.log(eps2) - jnp.log1p(-eps2)) * (1.0 / TMP))
    gn = jnp.concatenate(gn_l, axis=1)

    rowp = pad_halves(row, _H_USR)
    colp = pad_halves(col, 0)
    adjp = pad_halves(adj_vals, 0)
    rwrp = _pad_to(rw_row, ep_rw)
    rwcp = _pad_to(rw_col, ep_rw)
    rwvp = _pad_to(rw_vals, ep_rw)
    comboA = jnp.stack([rowp.reshape(-1, CHUNK), colp.reshape(-1, CHUNK)], axis=1)
    comboR = jnp.stack([rwrp.reshape(-1, CHUNK), rwcp.reshape(-1, CHUNK)], axis=1)

    w2s = edge_W2[:, :, 0] * (1.0 / TMP)
    b2s = jnp.tile(edge_b2 * (1.0 / TMP), (1, 16))
    zrows = jnp.zeros((_SR, D), jnp.float32)

    a0, b0, a1, b1, nmask, nsum = _prep(cur, edge_W1, edge_b1, node_W1,
                                        node_b1, node_W2, node_b2, gn)
    nv0, nv1, epart = _emask(a0, b0, a1, b1, rowp, colp, adjp,
                             ge_l[0], ge_l[1], w2s, b2s)

    e1 = _spmm_adj(orig, comboA, nv0, zrows)
    e2 = _spmm_adj(e1, comboA, nv1, zrows)

    mp1 = _spmm_rw(orig, comboR, rwvp, zrows)
    n1in = _combine(0, nmask, orig, mp1)
    n1 = _spmm_adj(n1in, comboA, adjp, zrows)
    mp2 = _spmm_rw(n1, comboR, rwvp, zrows)
    n2in = _combine(1, nmask, n1, mp2)
    n2 = _spmm_adj(n2in, comboA, adjp, zrows)

    oue, oip, oin, oun, onp_, onn, sreg = _final(
        user_id, pos_item, neg_item,
        orig, e1, e2, n1, n2, epart, nsum)
    return (oue, oip, oin, oun, onp_, onn, sreg[0], sreg[1])


# speedup vs baseline: 2.7259x; 1.0946x over previous
import functools

import jax
import jax.numpy as jnp
from jax import lax
from jax.experimental import pallas as pl
from jax.experimental.pallas import tpu as pltpu
from jax.experimental.pallas import tpu_sc as plsc

U_NUM = 2000
I_NUM = 8000
N_NODES = 10000
D = 128
TMP = 0.2
BIAS = 0.0001
E_ADJ = 320000
NC = 2
NS = 16
NW = NC * NS
CHUNK = 128


def _pad_to(x, n):
    return jnp.concatenate([x, jnp.zeros((n - x.shape[0],) + x.shape[1:], x.dtype)])


def _round_up(n, m):
    return ((n + m - 1) // m) * m


_BLK = 400


def _prep_body(cur_ref, ew1_ref, eb1_ref, nw1_ref, nb1_ref, nw2_ref, nb2_ref,
               gn_ref, a0_ref, b0_ref, a1_ref, b1_ref, nm_ref, ns_ref):
    x = cur_ref[...]
    pid = pl.program_id(0)

    @pl.when(pid == 0)
    def _():
        ns_ref[...] = jnp.zeros_like(ns_ref)

    sums = []
    for l in range(2):
        w = ew1_ref[l]
        a = jnp.dot(x, w[:D, :], preferred_element_type=jnp.float32) + eb1_ref[l][None, :]
        b = jnp.dot(x, w[D:, :], preferred_element_type=jnp.float32)
        if l == 0:
            a0_ref[...] = a
            b0_ref[...] = b
        else:
            a1_ref[...] = a
            b1_ref[...] = b
        h = jnp.maximum(
            jnp.dot(x, nw1_ref[l], preferred_element_type=jnp.float32) + nb1_ref[l][None, :], 0.0)
        nmv = jnp.dot(h, nw2_ref[l], preferred_element_type=jnp.float32) + nb2_ref[l][None, :]
        m = jax.nn.sigmoid(gn_ref[:, l:l + 1] + nmv * (1.0 / TMP))
        nm_ref[:, l:l + 1] = m
        sums.append(jnp.sum(m))
    lane = lax.broadcasted_iota(jnp.int32, (1, 128), 1)
    ns_ref[...] += jnp.where(lane == 0, sums[0], 0.0) + jnp.where(lane == 1, sums[1], 0.0)


def _prep(cur, ew1, eb1, nw1, nb1, nw2, nb2, gn):
    grid = N_NODES // _BLK

    def full(*s):
        return pl.BlockSpec(s, lambda i: tuple(0 for _ in s))

    return pl.pallas_call(
        _prep_body,
        grid=(grid,),
        in_specs=[
            pl.BlockSpec((_BLK, D), lambda i: (i, 0)),
            full(2, 2 * D, D), full(2, D), full(2, D, D), full(2, D),
            full(2, D, 1), full(2, 1),
            pl.BlockSpec((_BLK, 2), lambda i: (i, 0)),
        ],
        out_specs=[
            pl.BlockSpec((_BLK, D), lambda i: (i, 0)),
            pl.BlockSpec((_BLK, D), lambda i: (i, 0)),
            pl.BlockSpec((_BLK, D), lambda i: (i, 0)),
            pl.BlockSpec((_BLK, D), lambda i: (i, 0)),
            pl.BlockSpec((_BLK, 2), lambda i: (i, 0)),
            pl.BlockSpec((1, 128), lambda i: (0, 0)),
        ],
        out_shape=[
            jax.ShapeDtypeStruct((N_NODES, D), jnp.float32),
            jax.ShapeDtypeStruct((N_NODES, D), jnp.float32),
            jax.ShapeDtypeStruct((N_NODES, D), jnp.float32),
            jax.ShapeDtypeStruct((N_NODES, D), jnp.float32),
            jax.ShapeDtypeStruct((N_NODES, 2), jnp.float32),
            jax.ShapeDtypeStruct((1, 128), jnp.float32),
        ],
    )(cur, ew1, eb1, nw1, nb1, nw2, nb2, gn)


@functools.lru_cache(maxsize=None)
def _mesh():
    return plsc.VectorSubcoreMesh(core_axis_name="c", subcore_axis_name="s",
                                  num_cores=NC, num_subcores=NS)


_GDN = lax.GatherDimensionNumbers(offset_dims=(), collapsed_slice_dims=(0,),
                                  start_index_map=(0,))


def _shuf(v, idx16):
    return lax.gather(v, idx16[:, None], _GDN, (1,),
                      mode=lax.GatherScatterMode.PROMISE_IN_BOUNDS)


def _lanesum(v):
    lane = lax.broadcasted_iota(jnp.int32, (16,), 0)
    for sh in (8, 4, 2, 1):
        v = v + _shuf(v, lane ^ sh)
    return v


def _emask_body(a0, b0, a1, b1, comboA, comboM, w2s, b2s,
                nv0_out, nv1_out, epart_out,
                cb0, cb1, cm0, cm1, ar0, ar1, br0, br1, outv, w2v, b2v, partv,
                sa0, sa1, sb0, sb1):
    cid = lax.axis_index("c")
    sid = lax.axis_index("s")
    wid = sid * NC + cid
    nblk = comboA.shape[0]
    nch = nblk // NW
    base_blk = wid * nch
    pltpu.sync_copy(w2s, w2v)
    pltpu.sync_copy(b2s, b2v)
    lane = lax.broadcasted_iota(jnp.int32, (16,), 0)
    cb = [cb0, cb1]
    cm = [cm0, cm1]
    ar = [ar0, ar1]
    br = [br0, br1]
    sa = [sa0, sa1]
    sb = [sb0, sb1]
    for l in range(2):
        at = a0 if l == 0 else a1
        bt = b0 if l == 0 else b1
        nvl = nv0_out if l == 0 else nv1_out

        def load_fire(q, blk, at=at, bt=bt):
            pltpu.sync_copy(comboA.at[blk], cb[q])
            pltpu.sync_copy(comboM.at[blk], cm[q])
            pltpu.async_copy(at.at[cb[q].at[0]], ar[q], sa[q])
            pltpu.async_copy(bt.at[cb[q].at[1]], br[q], sb[q])

        def drain(q, at=at, bt=bt):
            pltpu.make_async_copy(at.at[cb[q].at[0]], ar[q], sa[q]).wait()
            pltpu.make_async_copy(bt.at[cb[q].at[1]], br[q], sb[q]).wait()

        def consume(q, c, reg, l=l, nvl=nvl):
            arq = ar[q]
            brq = br[q]
            cmq = cm[q]
            w2regs = [w2v[l, pl.ds(16 * k, 16)] for k in range(8)]
            b2reg = b2v[l]

            def grp_body(g, reg2):
                gb = g * 16
                dots = jnp.zeros((16,), jnp.float32)
                for e in range(16):
                    acc = jnp.zeros((16,), jnp.float32)
                    for k in range(8):
                        acc = acc + jnp.maximum(
                            arq[gb + e, pl.ds(16 * k, 16)]
                            + brq[gb + e, pl.ds(16 * k, 16)], 0.0) * w2regs[k]
                    dots = jnp.where(lane == e, _lanesum(acc), dots)
                g16 = cmq[1 + l, pl.ds(gb, 16)]
                ad16 = cmq[0, pl.ds(gb, 16)]
                m = 1.0 / (1.0 + jnp.exp(-(g16 + dots + b2reg)))
                nvv = ad16 * m
                outv[pl.ds(gb, 16)] = nvv
                return reg2 + nvv

            reg = lax.fori_loop(0, CHUNK // 16, grp_body, reg)
            pltpu.sync_copy(outv, nvl.at[pl.ds((base_blk + c) * CHUNK, CHUNK)])
            return reg

        load_fire(0, base_blk)

        def pair_body(i, reg):
            for q in (0, 1):
                c = 2 * i + q
                load_fire(1 - q, base_blk + c + 1)
                drain(q)
                reg = consume(q, c, reg)
            return reg

        reg = lax.fori_loop(0, (nch - 1) // 2, pair_body,
                            jnp.zeros((16,), jnp.float32))
        drain(0)
        reg = consume(0, nch - 1, reg)
        partv[0, ...] = reg
        pltpu.sync_copy(partv, epart_out.at[l * NW + wid])


def _emask(a0, b0, a1, b1, comboA, comboM, w2s, b2s):
    ep = comboA.shape[0] * CHUNK
    return pl.kernel(
        _emask_body,
        out_type=[
            jax.ShapeDtypeStruct((ep,), jnp.float32),
            jax.ShapeDtypeStruct((ep,), jnp.float32),
            jax.ShapeDtypeStruct((2 * NW, 1, 16), jnp.float32),
        ],
        mesh=_mesh(),
        scratch_types=[
            pltpu.VMEM((2, CHUNK), jnp.int32),
            pltpu.VMEM((2, CHUNK), jnp.int32),
            pltpu.VMEM((3, CHUNK), jnp.float32),
            pltpu.VMEM((3, CHUNK), jnp.float32),
            pltpu.VMEM((CHUNK, D), jnp.float32),
            pltpu.VMEM((CHUNK, D), jnp.float32),
            pltpu.VMEM((CHUNK, D), jnp.float32),
            pltpu.VMEM((CHUNK, D), jnp.float32),
            pltpu.VMEM((CHUNK,), jnp.float32),
            pltpu.VMEM((2, D), jnp.float32),
            pltpu.VMEM((2, 16), jnp.float32),
            pltpu.VMEM((1, 16), jnp.float32),
            pltpu.SemaphoreType.DMA,
            pltpu.SemaphoreType.DMA,
            pltpu.SemaphoreType.DMA,
            pltpu.SemaphoreType.DMA,
        ],
    )(a0, b0, a1, b1, comboA, comboM, w2s, b2s)


_SR = 640
_SR_LAST = N_NODES - _SR * (NS - 1)
_NACC = _SR * NS
_UACC = 8192
_H_USR = 2000


def _scale_rows(xb, vv):
    def grp_body(g, _):
        gb = g * 16
        v16 = vv[pl.ds(gb, 16)]
        for e in range(16):
            vb = v16[e]
            for k in range(8):
                xb[gb + e, pl.ds(16 * k, 16)] = xb[gb + e, pl.ds(16 * k, 16)] * vb
        return 0

    lax.fori_loop(0, CHUNK // 16, grp_body, 0)


def _spmm_pipe(combo, vals, x, acc, base_blk, nch, cb, vv, xb, sems, remap):
    def load_chunk(q, blk):
        pltpu.sync_copy(combo.at[blk], cb[q])
        pltpu.sync_copy(vals.at[pl.ds(blk * CHUNK, CHUNK)], vv[q])
        if remap is not None:
            for k in range(8):
                sl = pl.ds(16 * k, 16)
                cb[q][0, sl] = cb[q][0, sl] - remap

    def fire(q):
        pltpu.async_copy(x.at[cb[q].at[1]], xb[q], sems[q])

    def drain(q):
        pltpu.make_async_copy(x.at[cb[q].at[1]], xb[q], sems[q]).wait()

    def consume(q):
        _scale_rows(xb[q], vv[q])
        pltpu.sync_copy(xb[q], acc.at[cb[q].at[0]], add=True)

    load_chunk(0, base_blk)
    fire(0)

    def pair_body(i, _):
        for q in (0, 1):
            c = 2 * i + q
            load_chunk(1 - q, base_blk + c + 1)
            fire(1 - q)
            drain(q)
            consume(q)
        return 0

    lax.fori_loop(0, (nch - 1) // 2, pair_body, 0)
    drain(0)
    consume(0)


def _spmm_adj_body(x, combo, vals, zrows, out,
                   cb0, cb1, vv0, vv1, xb0, xb1, acc, s0, s1):
    cid = lax.axis_index("c")
    sid = lax.axis_index("s")
    ep = vals.shape[0]
    ept = ep // NW
    nch = ept // CHUNK
    base_blk = (cid * (ep // 2) + sid * ept) // CHUNK

    @pl.when(cid == 0)
    def _():
        pltpu.sync_copy(zrows.at[pl.ds(0, 128)], acc.at[pl.ds(sid * 128, 128)])

    @pl.when(cid == 1)
    def _():
        pltpu.sync_copy(zrows.at[pl.ds(0, 512)], acc.at[pl.ds(sid * 512, 512)])

    plsc.subcore_barrier()
    _spmm_pipe(combo, vals, x, acc, base_blk, nch,
               [cb0, cb1], [vv0, vv1], [xb0, xb1], [s0, s1],
               remap=cid * _H_USR)
    plsc.subcore_barrier()

    @pl.when(cid == 0)
    def _():
        @pl.when(sid < NS - 1)
        def _():
            pltpu.sync_copy(acc.at[pl.ds(sid * 128, 128)],
                            out.at[pl.ds(sid * 128, 128)])

        @pl.when(sid == NS - 1)
        def _():
            pltpu.sync_copy(acc.at[pl.ds(1920, 80)], out.at[pl.ds(1920, 80)])

    @pl.when(cid == 1)
    def _():
        @pl.when(sid < NS - 1)
        def _():
            pltpu.sync_copy(acc.at[pl.ds(sid * 512, 512)],
                            out.at[pl.ds(_H_USR + sid * 512, 512)])

        @pl.when(sid == NS - 1)
        def _():
            pltpu.sync_copy(acc.at[pl.ds(7680, 320)], out.at[pl.ds(9680, 320)])


def _spmm_adj(x, combo, vals, zrows):
    return pl.kernel(
        _spmm_adj_body,
        out_type=[jax.ShapeDtypeStruct((N_NODES, D), jnp.float32)],
        mesh=_mesh(),
        scratch_types=[
            pltpu.VMEM((2, CHUNK), jnp.int32),
            pltpu.VMEM((2, CHUNK), jnp.int32),
            pltpu.VMEM((CHUNK,), jnp.float32),
            pltpu.VMEM((CHUNK,), jnp.float32),
            pltpu.VMEM((CHUNK, D), jnp.float32),
            pltpu.VMEM((CHUNK, D), jnp.float32),
            pltpu.VMEM_SHARED((_UACC, D), jnp.float32),
            pltpu.SemaphoreType.DMA,
            pltpu.SemaphoreType.DMA,
        ],
    )(x, combo, vals, zrows)[0]


def _spmm_rw_body(x, combo, vals, zrows, part,
                  cb0, cb1, vv0, vv1, xb0, xb1, acc, s0, s1):
    cid = lax.axis_index("c")
    sid = lax.axis_index("s")
    wid = sid * NC + cid
    ep = vals.shape[0]
    ept = ep // NW
    nch = ept // CHUNK
    base_blk = wid * ept // CHUNK
    rbase = sid * _SR

    @pl.when(sid < NS - 1)
    def _():
        pltpu.sync_copy(zrows, acc.at[pl.ds(rbase, _SR)])

    @pl.when(sid == NS - 1)
    def _():
        pltpu.sync_copy(zrows.at[pl.ds(0, _SR_LAST)], acc.at[pl.ds(rbase, _SR_LAST)])

    plsc.subcore_barrier()
    _spmm_pipe(combo, vals, x, acc, base_blk, nch,
               [cb0, cb1], [vv0, vv1], [xb0, xb1], [s0, s1], remap=None)
    plsc.subcore_barrier()

    @pl.when(sid < NS - 1)
    def _():
        pltpu.sync_copy(acc.at[pl.ds(rbase, _SR)], part.at[cid, pl.ds(rbase, _SR)])

    @pl.when(sid == NS - 1)
    def _():
        pltpu.sync_copy(acc.at[pl.ds(rbase, _SR_LAST)],
                        part.at[cid, pl.ds(rbase, _SR_LAST)])


def _spmm_rw(x, combo, vals, zrows):
    return pl.kernel(
        _spmm_rw_body,
        out_type=[jax.ShapeDtypeStruct((NC, N_NODES, D), jnp.float32)],
        mesh=_mesh(),
        scratch_types=[
            pltpu.VMEM((2, CHUNK), jnp.int32),
            pltpu.VMEM((2, CHUNK), jnp.int32),
            pltpu.VMEM((CHUNK,), jnp.float32),
            pltpu.VMEM((CHUNK,), jnp.float32),
            pltpu.VMEM((CHUNK, D), jnp.float32),
            pltpu.VMEM((CHUNK, D), jnp.float32),
            pltpu.VMEM_SHARED((_NACC, D), jnp.float32),
            pltpu.SemaphoreType.DMA,
            pltpu.SemaphoreType.DMA,
        ],
    )(x, combo, vals, zrows)[0]


def _combine_body(l, nm_ref, cur_ref, p0_ref, p1_ref, o_ref):
    m = nm_ref[:, l:l + 1]
    o_ref[...] = m * cur_ref[...] + (1.0 - m) * (p0_ref[...] + p1_ref[...])


def _combine(l, nm, cur, mp):
    return pl.pallas_call(
        functools.partial(_combine_body, l),
        grid=(N_NODES // _BLK,),
        in_specs=[
            pl.BlockSpec((_BLK, 2), lambda i: (i, 0)),
            pl.BlockSpec((_BLK, D), lambda i: (i, 0)),
            pl.BlockSpec((_BLK, D), lambda i: (i, 0)),
            pl.BlockSpec((_BLK, D), lambda i: (i, 0)),
        ],
        out_specs=pl.BlockSpec((_BLK, D), lambda i: (i, 0)),
        out_shape=jax.ShapeDtypeStruct((N_NODES, D), jnp.float32),
    )(nm, cur, mp[0], mp[1])


_BQ = 1024
_BT = _BQ // NW


def _fin_body(uid, pid_, nid, e0, e1, e2, n1, n2, epart, nsum,
              oue, oip, oin, oun, onp, onn, sreg,
              idxv, r0, r1, r2, ob, epbuf, nsbuf, sbuf,
              s0, s1, s2):
    cid = lax.axis_index("c")
    sid = lax.axis_index("s")
    wid = sid * NC + cid
    base = wid * _BT
    third = 1.0 / 3.0

    def gather_combine(ids_hbm, tabs, out_hbm, offset):
        pltpu.sync_copy(ids_hbm.at[pl.ds(base, _BT)], idxv)
        if offset:
            for j in range(_BT // 16):
                idxv[pl.ds(16 * j, 16)] = idxv[pl.ds(16 * j, 16)] + U_NUM
        c0 = pltpu.async_copy(tabs[0].at[idxv], r0, s0)
        c1 = pltpu.async_copy(tabs[1].at[idxv], r1, s1)
        c2 = pltpu.async_copy(tabs[2].at[idxv], r2, s2)
        c0.wait()
        c1.wait()
        c2.wait()
        def row_body(j, _):
            for k in range(8):
                sl = pl.ds(16 * k, 16)
                ob[j, sl] = (r0[j, sl] + r1[j, sl] + r2[j, sl]) * third
            return 0

        lax.fori_loop(0, _BT, row_body, 0)
        pltpu.sync_copy(ob, out_hbm.at[pl.ds(base, _BT)])

    etabs = (e0, e1, e2)
    ntabs = (e0, n1, n2)
    gather_combine(uid, etabs, oue, False)
    gather_combine(uid, ntabs, oun, False)
    gather_combine(pid_, etabs, oip, True)
    gather_combine(pid_, ntabs, onp, True)
    gather_combine(nid, etabs, oin, True)
    gather_combine(nid, ntabs, onn, True)

    @pl.when(wid == 0)
    def _():
        pltpu.sync_copy(epart, epbuf)
        pltpu.sync_copy(nsum, nsbuf)

        def rb(i, a):
            return a + epbuf[i, 0]

        acc = lax.fori_loop(0, 2 * NW, rb, jnp.zeros((16,), jnp.float32))
        er = _lanesum(acc) * (1.0 / (E_ADJ * 2))
        lane = lax.broadcasted_iota(jnp.int32, (16,), 0)
        nr = _lanesum(jnp.where(lane < 2, nsbuf[0, pl.ds(0, 16)], 0.0)) * (1.0 / (N_NODES * 2))
        sbuf[...] = jnp.where(lane == 0, er, jnp.where(lane == 1, nr, 0.0))
        pltpu.sync_copy(sbuf, sreg)


def _final(uid, pid_, nid, e0, e1, e2, n1, n2, epart, nsum):
    obt = jax.ShapeDtypeStruct((_BQ, D), jnp.float32)
    return pl.kernel(
        _fin_body,
        out_type=[obt, obt, obt, obt, obt, obt,
                  jax.ShapeDtypeStruct((16,), jnp.float32)],
        mesh=_mesh(),
        scratch_types=[
            pltpu.VMEM((_BT,), jnp.int32),
            pltpu.VMEM((_BT, D), jnp.float32),
            pltpu.VMEM((_BT, D), jnp.float32),
            pltpu.VMEM((_BT, D), jnp.float32),
            pltpu.VMEM((_BT, D), jnp.float32),
            pltpu.VMEM((2 * NW, 1, 16), jnp.float32),
            pltpu.VMEM((1, 128), jnp.float32),
            pltpu.VMEM((16,), jnp.float32),
            pltpu.SemaphoreType.DMA,
            pltpu.SemaphoreType.DMA,
            pltpu.SemaphoreType.DMA,
        ],
    )(uid, pid_, nid, e0, e1, e2, n1, n2, epart, nsum)


def kernel(cur_user_emb, cur_item_emb, original_user_emb, original_item_emb,
           user_id, pos_item, neg_item,
           row, col, adj_vals, rw_row, rw_col, rw_vals,
           node_W1, node_b1, node_W2, node_b2,
           edge_W1, edge_b1, edge_W2, edge_b2):
    cur = jnp.concatenate([cur_user_emb, cur_item_emb], axis=0)
    orig = jnp.concatenate([original_user_emb, original_item_emb], axis=0)

    e_rw = rw_row.shape[0]
    ep_adj = _round_up(E_ADJ, NW * CHUNK)
    ep_rw = _round_up(e_rw, NW * CHUNK)
    if (ep_rw // NW // CHUNK) % 2 == 0:
        ep_rw += NW * CHUNK

    hh = E_ADJ // 2
    hp = ep_adj // 2
    padu = hp - hh

    def pad_halves(arr, pv1):
        z0 = jnp.zeros((padu,), arr.dtype)
        z1 = jnp.full((padu,), pv1, arr.dtype)
        return jnp.concatenate([arr[:hh], z0, arr[hh:], z1])

    key = jax.random.key(42)
    ge_l, gn_l = [], []
    for l in range(2):
        u = jax.random.uniform(jax.random.fold_in(key, 2 * l), (E_ADJ, 1))
        eps = (BIAS - (1 - BIAS)) * u + (1 - BIAS)
        ge_l.append(pad_halves((jnp.log(eps) - jnp.log1p(-eps))[:, 0] * (1.0 / TMP), 0))
        u2 = jax.random.uniform(jax.random.fold_in(key, 2 * l + 1), (N_NODES, 1))
        eps2 = (BIAS - (1 - BIAS)) * u2 + (1 - BIAS)
        gn_l.append((jnp.log(eps2) - jnp.log1p(-eps2)) * (1.0 / TMP))
    gn = jnp.concatenate(gn_l, axis=1)

    rowp = pad_halves(row, _H_USR)
    colp = pad_halves(col, 0)
    adjp = pad_halves(adj_vals, 0)
    rwrp = _pad_to(rw_row, ep_rw)
    rwcp = _pad_to(rw_col, ep_rw)
    rwvp = _pad_to(rw_vals, ep_rw)
    comboA = jnp.stack([rowp.reshape(-1, CHUNK), colp.reshape(-1, CHUNK)], axis=1)
    comboM = jnp.stack([adjp.reshape(-1, CHUNK), ge_l[0].reshape(-1, CHUNK),
                        ge_l[1].reshape(-1, CHUNK)], axis=1)
    comboR = jnp.stack([rwrp.reshape(-1, CHUNK), rwcp.reshape(-1, CHUNK)], axis=1)

    w2s = edge_W2[:, :, 0] * (1.0 / TMP)
    b2s = jnp.tile(edge_b2 * (1.0 / TMP), (1, 16))
    zrows = jnp.zeros((_SR, D), jnp.float32)

    a0, b0, a1, b1, nmask, nsum = _prep(cur, edge_W1, edge_b1, node_W1,
                                        node_b1, node_W2, node_b2, gn)
    nv0, nv1, epart = _emask(a0, b0, a1, b1, comboA, comboM, w2s, b2s)

    e1 = _spmm_adj(orig, comboA, nv0, zrows)
    e2 = _spmm_adj(e1, comboA, nv1, zrows)

    mp1 = _spmm_rw(orig, comboR, rwvp, zrows)
    n1in = _combine(0, nmask, orig, mp1)
    n1 = _spmm_adj(n1in, comboA, adjp, zrows)
    mp2 = _spmm_rw(n1, comboR, rwvp, zrows)
    n2in = _combine(1, nmask, n1, mp2)
    n2 = _spmm_adj(n2in, comboA, adjp, zrows)

    oue, oip, oin, oun, onp_, onn, sreg = _final(
        user_id, pos_item, neg_item,
        orig, e1, e2, n1, n2, epart, nsum)
    return (oue, oip, oin, oun, onp_, onn, sreg[0], sreg[1])
